# Initial kernel scaffold; baseline (speedup 1.0000x reference)
import jax
import jax.numpy as jnp
from jax.experimental import pallas as pl


def kernel(x, patch_W, patch_b, cls_token, pos_embed, qkv_W, qkv_b, out_W, out_b, ln1_g, ln1_b, ln2_g, ln2_b, router_W, router_b, e_W1, e_b1, e_W2, e_b2, fn_g, fn_b, fc_W, fc_b):
    raise NotImplementedError("write your pallas kernel here")


# speedup vs baseline: 1.8342x; 1.8342x over previous
import functools
import math

import jax
import jax.numpy as jnp
from jax import lax
from jax.experimental import pallas as pl
from jax.experimental.pallas import tpu as pltpu
from jax.experimental.pallas import tpu_sc as plsc

P = 16
NH = 12
NE = 8
D = 384
HD = 1536
S = 197
B = 8
NTOK = B * S
NW = 32
BPW = 56
NPAD = NW * BPW
CHUNK = 128
NCHUNK = NPAD // CHUNK


def _gelu(x):
    return 0.5 * x * (1.0 + lax.erf(x / jnp.sqrt(jnp.float32(2.0))))


def _ln(x, g, b):
    m = jnp.mean(x, axis=-1, keepdims=True)
    v = jnp.mean((x - m) ** 2, axis=-1, keepdims=True)
    return (x - m) / jnp.sqrt(v + 1e-5) * g + b


def _embed_body(p_ref, w_ref, b_ref, cls_ref, pos_ref, o_ref):
    p = p_ref[0]
    t = lax.dot_general(p, w_ref[...], (((1,), (1,)), ((), ())),
                        preferred_element_type=jnp.float32)
    t = t + b_ref[...] + pos_ref[1:, :]
    o_ref[0, 0:1, :] = cls_ref[...] + pos_ref[0:1, :]
    o_ref[0, 1:, :] = t


def _embed_call(patches, patch_W, patch_b, cls_token, pos_embed):
    nb = patches.shape[0]
    return pl.pallas_call(
        _embed_body,
        grid=(nb,),
        in_specs=[
            pl.BlockSpec((1, 196, 768), lambda i: (i, 0, 0)),
            pl.BlockSpec((D, 768), lambda i: (0, 0)),
            pl.BlockSpec((1, D), lambda i: (0, 0)),
            pl.BlockSpec((1, D), lambda i: (0, 0)),
            pl.BlockSpec((S, D), lambda i: (0, 0)),
        ],
        out_specs=pl.BlockSpec((1, S, D), lambda i: (i, 0, 0)),
        out_shape=jax.ShapeDtypeStruct((nb, S, D), jnp.float32),
    )(patches, patch_W, patch_b.reshape(1, D), cls_token.reshape(1, D),
      pos_embed.reshape(S, D))


def _attn_body(h_ref, wqkv_ref, bqkv_ref, wo_ref, bo_ref, g_ref, be_ref,
               o_ref):
    x = h_ref[0]
    qkv = lax.dot_general(x, wqkv_ref[...], (((1,), (1,)), ((), ())),
                          preferred_element_type=jnp.float32) + bqkv_ref[...]
    hd = D // NH
    scale = jnp.sqrt(jnp.float32(hd))
    heads = []
    for i in range(NH):
        q = qkv[:, i * hd:(i + 1) * hd]
        k = qkv[:, D + i * hd:D + (i + 1) * hd]
        v = qkv[:, 2 * D + i * hd:2 * D + (i + 1) * hd]
        s = lax.dot_general(q, k, (((1,), (1,)), ((), ())),
                            preferred_element_type=jnp.float32) / scale
        m = jnp.max(s, axis=1, keepdims=True)
        e = jnp.exp(s - m)
        a = e / jnp.sum(e, axis=1, keepdims=True)
        heads.append(lax.dot_general(a, v, (((1,), (0,)), ((), ())),
                                     preferred_element_type=jnp.float32))
    o = jnp.concatenate(heads, axis=1)
    ao = lax.dot_general(o, wo_ref[...], (((1,), (1,)), ((), ())),
                         preferred_element_type=jnp.float32) + bo_ref[...]
    o_ref[0] = _ln(x + ao, g_ref[...], be_ref[...])


def _attn_call(h, wqkv, bqkv, wo, bo, g, be):
    return pl.pallas_call(
        _attn_body,
        grid=(B,),
        in_specs=[
            pl.BlockSpec((1, S, D), lambda i: (i, 0, 0)),
            pl.BlockSpec((3 * D, D), lambda i: (0, 0)),
            pl.BlockSpec((1, 3 * D), lambda i: (0, 0)),
            pl.BlockSpec((D, D), lambda i: (0, 0)),
            pl.BlockSpec((1, D), lambda i: (0, 0)),
            pl.BlockSpec((1, D), lambda i: (0, 0)),
            pl.BlockSpec((1, D), lambda i: (0, 0)),
        ],
        out_specs=pl.BlockSpec((1, S, D), lambda i: (i, 0, 0)),
        out_shape=jax.ShapeDtypeStruct((B, S, D), jnp.float32),
    )(h, wqkv, bqkv.reshape(1, 3 * D), wo, bo.reshape(1, D),
      g.reshape(1, D), be.reshape(1, D))


def _route_body(xf_ref, rw_ref, rb_ref, nz_ref, dest_ref, sinfo_ref):
    logits = lax.dot_general(xf_ref[...], rw_ref[...],
                             (((1,), (1,)), ((), ())),
                             preferred_element_type=jnp.float32)
    logits = logits + rb_ref[...] + nz_ref[...]
    mx = jnp.max(logits, axis=1, keepdims=True)
    lane = lax.broadcasted_iota(jnp.int32, (NTOK, NE), 1)
    eidx = jnp.min(jnp.where(logits == mx, lane, NE), axis=1, keepdims=True)
    onehot = (lane == eidx).astype(jnp.float32)
    counts = jnp.sum(onehot, axis=0, keepdims=True)
    r8 = lax.broadcasted_iota(jnp.int32, (NE, NE), 0)
    c8 = lax.broadcasted_iota(jnp.int32, (NE, NE), 1)
    tstrict = (r8 < c8).astype(jnp.float32)
    starts = lax.dot_general(counts, tstrict, (((1,), (0,)), ((), ())),
                             preferred_element_type=jnp.float32)
    cum = onehot
    shift = 1
    while shift < NTOK:
        z = jnp.zeros((shift, NE), jnp.float32)
        cum = cum + jnp.concatenate([z, cum[:NTOK - shift, :]], axis=0)
        shift *= 2
    rank = jnp.sum(cum * onehot, axis=1, keepdims=True)
    sbase = jnp.sum(starts * onehot, axis=1, keepdims=True)
    dest_ref[...] = (sbase + rank - 1.0).astype(jnp.int32)
    ends = starts + counts
    sinfo_ref[...] = jnp.concatenate([starts, ends], axis=1).astype(jnp.int32)


def _route_call(xf, rw, rb, noise):
    return pl.pallas_call(
        _route_body,
        out_shape=(jax.ShapeDtypeStruct((NTOK, 1), jnp.int32),
                   jax.ShapeDtypeStruct((1, 2 * NE), jnp.int32)),
    )(xf, rw, rb.reshape(1, NE), noise)


def _ffn_body(sinfo_ref, x_ref, w1_ref, b1_ref, w2_ref, b2_ref, o_ref):
    j = pl.program_id(0)
    w1 = w1_ref[0]
    w2 = w2_ref[0]
    b1 = b1_ref[0]
    b2 = b2_ref[0]
    cj = lax.dot_general(_gelu(b1), w2, (((1,), (1,)), ((), ())),
                         preferred_element_type=jnp.float32) + b2

    @pl.when(j == 0)
    def _():
        o_ref[...] = jnp.broadcast_to(cj, (NPAD, D))

    @pl.when(j > 0)
    def _():
        o_ref[...] = o_ref[...] + cj

    s = sinfo_ref[j]
    e = sinfo_ref[NE + j]
    for c in range(NCHUNK):
        base = c * CHUNK

        @pl.when((e > base) & (s < base + CHUNK))
        def _():
            xb = x_ref[base:base + CHUNK, :]
            h1 = _gelu(lax.dot_general(xb, w1, (((1,), (1,)), ((), ())),
                                       preferred_element_type=jnp.float32)
                       + b1)
            y = lax.dot_general(h1, w2, (((1,), (1,)), ((), ())),
                                preferred_element_type=jnp.float32) + b2
            rows = base + lax.broadcasted_iota(jnp.int32, (CHUNK, 1), 0)
            msk = (rows >= s) & (rows < e)
            o_ref[base:base + CHUNK, :] = (
                o_ref[base:base + CHUNK, :]
                + jnp.where(msk, y - cj, jnp.float32(0.0)))


def _ffn_call(sinfo, sorted_x, w1, b1, w2, b2):
    return pl.pallas_call(
        _ffn_body,
        grid=(NE,),
        in_specs=[
            pl.BlockSpec(memory_space=pltpu.SMEM),
            pl.BlockSpec((NPAD, D), lambda j: (0, 0)),
            pl.BlockSpec((1, HD, D), lambda j: (j, 0, 0)),
            pl.BlockSpec((1, 1, HD), lambda j: (j, 0, 0)),
            pl.BlockSpec((1, D, HD), lambda j: (j, 0, 0)),
            pl.BlockSpec((1, 1, D), lambda j: (j, 0, 0)),
        ],
        out_specs=pl.BlockSpec((NPAD, D), lambda j: (0, 0)),
        out_shape=jax.ShapeDtypeStruct((NPAD, D), jnp.float32),
    )(sinfo, sorted_x, w1, b1.reshape(NE, 1, HD), w2, b2.reshape(NE, 1, D))


@functools.lru_cache(maxsize=None)
def _sc_kernels():
    mesh = plsc.VectorSubcoreMesh(core_axis_name="c", subcore_axis_name="s")
    kern = functools.partial(
        pl.kernel, mesh=mesh,
        out_type=jax.ShapeDtypeStruct((NPAD, D), jnp.float32),
        scratch_types=[
            pltpu.VMEM((BPW,), jnp.int32),
            pltpu.VMEM((BPW, D), jnp.float32),
            pltpu.SemaphoreType.DMA,
        ],
    )

    @kern
    def sc_scatter(x_hbm, idx_hbm, out_hbm, idx_v, rows_v, sem):
        wid = lax.axis_index("s") * 2 + lax.axis_index("c")
        base = wid * BPW
        pltpu.sync_copy(idx_hbm.at[pl.ds(base, BPW)], idx_v)
        pltpu.sync_copy(x_hbm.at[pl.ds(base, BPW)], rows_v)
        pltpu.async_copy(rows_v, out_hbm.at[idx_v], sem).wait()

    @kern
    def sc_gather(y_hbm, idx_hbm, out_hbm, idx_v, rows_v, sem):
        wid = lax.axis_index("s") * 2 + lax.axis_index("c")
        base = wid * BPW
        pltpu.sync_copy(idx_hbm.at[pl.ds(base, BPW)], idx_v)
        pltpu.async_copy(y_hbm.at[idx_v], rows_v, sem).wait()
        pltpu.sync_copy(rows_v, out_hbm.at[pl.ds(base, BPW)])

    return sc_scatter, sc_gather


def _resln_body(h_ref, y_ref, g_ref, b_ref, o_ref):
    o_ref[...] = _ln(h_ref[...] + y_ref[...], g_ref[...], b_ref[...])


def _resln_call(h, y, g, b):
    return pl.pallas_call(
        _resln_body,
        out_shape=jax.ShapeDtypeStruct((B, S, D), jnp.float32),
    )(h, y, g.reshape(1, 1, D), b.reshape(1, 1, D))


def _head_body(h_ref, g_ref, b_ref, wf_ref, bf_ref, o_ref):
    x = h_ref[:, 0, :]
    x = _ln(x, g_ref[...], b_ref[...])
    o_ref[...] = lax.dot_general(x, wf_ref[...], (((1,), (1,)), ((), ())),
                                 preferred_element_type=jnp.float32) \
        + bf_ref[...]


def _head_call(h, g, b, wf, bf):
    nc = wf.shape[0]
    return pl.pallas_call(
        _head_body,
        out_shape=jax.ShapeDtypeStruct((B, nc), jnp.float32),
    )(h, g.reshape(1, D), b.reshape(1, D), wf, bf.reshape(1, nc))


def kernel(x, patch_W, patch_b, cls_token, pos_embed, qkv_W, qkv_b, out_W,
           out_b, ln1_g, ln1_b, ln2_g, ln2_b, router_W, router_b, e_W1, e_b1,
           e_W2, e_b2, fn_g, fn_b, fc_W, fc_b):
    nb, C, H, W = x.shape
    hp, wp = H // P, W // P
    patches = x.reshape(nb, C, hp, P, wp, P).transpose(0, 1, 2, 4, 3, 5)
    patches = patches.reshape(nb, C, hp * wp, P * P).transpose(0, 2, 1, 3)
    patches = patches.reshape(nb, hp * wp, C * P * P)

    h = _embed_call(patches, patch_W, patch_b, cls_token, pos_embed)

    L = qkv_W.shape[0]
    nkey = jax.random.key(42)
    pad_idx = jnp.arange(NTOK, NPAD, dtype=jnp.int32)
    sc_scatter, sc_gather = _sc_kernels()
    for l in range(L):
        h = _attn_call(h, qkv_W[l], qkv_b[l], out_W[l], out_b[l],
                       ln1_g[l], ln1_b[l])
        xf = h.reshape(NTOK, D)
        noise = jax.random.normal(jax.random.fold_in(nkey, l), (NTOK, NE),
                                  dtype=jnp.float32) * 0.01
        dest, sinfo = _route_call(xf, router_W[l], router_b[l], noise)
        dest_pad = jnp.concatenate([dest[:, 0], pad_idx])
        xf_pad = jnp.pad(xf, ((0, NPAD - NTOK), (0, 0)))
        sorted_x = sc_scatter(xf_pad, dest_pad)
        sorted_y = _ffn_call(sinfo.reshape(2 * NE), sorted_x,
                             e_W1[l], e_b1[l], e_W2[l], e_b2[l])
        y = sc_gather(sorted_y, dest_pad)
        h = _resln_call(h, y[:NTOK].reshape(B, S, D), ln2_g[l], ln2_b[l])

    return _head_call(h, fn_g, fn_b, fc_W, fc_b)


# speedup vs baseline: 1.9630x; 1.0702x over previous
import functools
import math

import jax
import jax.numpy as jnp
from jax import lax
from jax.experimental import pallas as pl
from jax.experimental.pallas import tpu as pltpu
from jax.experimental.pallas import tpu_sc as plsc

P = 16
NH = 12
NE = 8
D = 384
HD = 1536
S = 197
B = 8
NTOK = B * S
NW = 32
BPW = 56
NPAD = NW * BPW
CHUNK = 128
NCHUNK = NPAD // CHUNK


def _gelu(x):
    return 0.5 * x * (1.0 + lax.erf(x / jnp.sqrt(jnp.float32(2.0))))


def _ln(x, g, b):
    m = jnp.mean(x, axis=-1, keepdims=True)
    v = jnp.mean((x - m) ** 2, axis=-1, keepdims=True)
    return (x - m) / jnp.sqrt(v + 1e-5) * g + b


def _embed_body(p_ref, w_ref, b_ref, cls_ref, pos_ref, o_ref):
    p = p_ref[0]
    t = lax.dot_general(p, w_ref[...], (((1,), (1,)), ((), ())),
                        preferred_element_type=jnp.float32)
    t = t + b_ref[...] + pos_ref[1:, :]
    o_ref[0, 0:1, :] = cls_ref[...] + pos_ref[0:1, :]
    o_ref[0, 1:, :] = t


def _embed_call(patches, patch_W, patch_b, cls_token, pos_embed):
    nb = patches.shape[0]
    return pl.pallas_call(
        _embed_body,
        grid=(nb,),
        in_specs=[
            pl.BlockSpec((1, 196, 768), lambda i: (i, 0, 0)),
            pl.BlockSpec((D, 768), lambda i: (0, 0)),
            pl.BlockSpec((1, D), lambda i: (0, 0)),
            pl.BlockSpec((1, D), lambda i: (0, 0)),
            pl.BlockSpec((S, D), lambda i: (0, 0)),
        ],
        out_specs=pl.BlockSpec((1, S, D), lambda i: (i, 0, 0)),
        out_shape=jax.ShapeDtypeStruct((nb, S, D), jnp.float32),
    )(patches, patch_W, patch_b.reshape(1, D), cls_token.reshape(1, D),
      pos_embed.reshape(S, D))


def _attn_body(h_ref, wqkv_ref, bqkv_ref, wo_ref, bo_ref, g_ref, be_ref,
               o_ref):
    x = h_ref[0]
    qkv = lax.dot_general(x, wqkv_ref[...], (((1,), (1,)), ((), ())),
                          preferred_element_type=jnp.float32) + bqkv_ref[...]
    hd = D // NH
    scale = jnp.sqrt(jnp.float32(hd))
    heads = []
    for i in range(NH):
        q = qkv[:, i * hd:(i + 1) * hd]
        k = qkv[:, D + i * hd:D + (i + 1) * hd]
        v = qkv[:, 2 * D + i * hd:2 * D + (i + 1) * hd]
        s = lax.dot_general(q, k, (((1,), (1,)), ((), ())),
                            preferred_element_type=jnp.float32) / scale
        m = jnp.max(s, axis=1, keepdims=True)
        e = jnp.exp(s - m)
        a = e / jnp.sum(e, axis=1, keepdims=True)
        heads.append(lax.dot_general(a, v, (((1,), (0,)), ((), ())),
                                     preferred_element_type=jnp.float32))
    o = jnp.concatenate(heads, axis=1)
    ao = lax.dot_general(o, wo_ref[...], (((1,), (1,)), ((), ())),
                         preferred_element_type=jnp.float32) + bo_ref[...]
    o_ref[0] = _ln(x + ao, g_ref[...], be_ref[...])


def _attn_call(h, wqkv, bqkv, wo, bo, g, be):
    return pl.pallas_call(
        _attn_body,
        grid=(B,),
        in_specs=[
            pl.BlockSpec((1, S, D), lambda i: (i, 0, 0)),
            pl.BlockSpec((3 * D, D), lambda i: (0, 0)),
            pl.BlockSpec((1, 3 * D), lambda i: (0, 0)),
            pl.BlockSpec((D, D), lambda i: (0, 0)),
            pl.BlockSpec((1, D), lambda i: (0, 0)),
            pl.BlockSpec((1, D), lambda i: (0, 0)),
            pl.BlockSpec((1, D), lambda i: (0, 0)),
        ],
        out_specs=pl.BlockSpec((1, S, D), lambda i: (i, 0, 0)),
        out_shape=jax.ShapeDtypeStruct((B, S, D), jnp.float32),
    )(h, wqkv, bqkv.reshape(1, 3 * D), wo, bo.reshape(1, D),
      g.reshape(1, D), be.reshape(1, D))


def _route_body(xf_ref, rw_ref, rb_ref, nz_ref, dest_ref, sinfo_ref,
                sx_ref):
    logits = lax.dot_general(xf_ref[...], rw_ref[...],
                             (((1,), (1,)), ((), ())),
                             preferred_element_type=jnp.float32)
    logits = logits + rb_ref[...] + nz_ref[...]
    mx = jnp.max(logits, axis=1, keepdims=True)
    lane = lax.broadcasted_iota(jnp.int32, (NTOK, NE), 1)
    eidx = jnp.min(jnp.where(logits == mx, lane, NE), axis=1, keepdims=True)
    onehot = (lane == eidx).astype(jnp.float32)
    counts = jnp.sum(onehot, axis=0, keepdims=True)
    r8 = lax.broadcasted_iota(jnp.int32, (NE, NE), 0)
    c8 = lax.broadcasted_iota(jnp.int32, (NE, NE), 1)
    tstrict = (r8 < c8).astype(jnp.float32)
    starts = lax.dot_general(counts, tstrict, (((1,), (0,)), ((), ())),
                             preferred_element_type=jnp.float32)
    cum = onehot
    shift = 1
    while shift < NTOK:
        z = jnp.zeros((shift, NE), jnp.float32)
        cum = cum + jnp.concatenate([z, cum[:NTOK - shift, :]], axis=0)
        shift *= 2
    rank = jnp.sum(cum * onehot, axis=1, keepdims=True)
    sbase = jnp.sum(starts * onehot, axis=1, keepdims=True)
    dest = (sbase + rank - 1.0).astype(jnp.int32)
    dest_ref[...] = dest
    ends = starts + counts
    sinfo_ref[...] = jnp.concatenate([starts, ends], axis=1).astype(jnp.int32)
    pos = lax.broadcasted_iota(jnp.int32, (NTOK, NPAD), 1)
    pt = (pos == dest).astype(jnp.float32)
    sx_ref[...] = lax.dot_general(pt, xf_ref[...], (((0,), (0,)), ((), ())),
                                  preferred_element_type=jnp.float32)


def _route_call(xf, rw, rb, noise):
    return pl.pallas_call(
        _route_body,
        out_shape=(jax.ShapeDtypeStruct((NTOK, 1), jnp.int32),
                   jax.ShapeDtypeStruct((1, 2 * NE), jnp.int32),
                   jax.ShapeDtypeStruct((NPAD, D), jnp.float32)),
    )(xf, rw, rb.reshape(1, NE), noise)


def _ffn_body(sinfo_ref, x_ref, w1_ref, b1_ref, w2_ref, b2_ref, o_ref):
    j = pl.program_id(0)
    w1 = w1_ref[0]
    w2 = w2_ref[0]
    b1 = b1_ref[0]
    b2 = b2_ref[0]
    cj = lax.dot_general(_gelu(b1), w2, (((1,), (1,)), ((), ())),
                         preferred_element_type=jnp.float32) + b2

    @pl.when(j == 0)
    def _():
        o_ref[...] = jnp.broadcast_to(cj, (NPAD, D))

    @pl.when(j > 0)
    def _():
        o_ref[...] = o_ref[...] + cj

    s = sinfo_ref[j]
    e = sinfo_ref[NE + j]
    for c in range(NCHUNK):
        base = c * CHUNK

        @pl.when((e > base) & (s < base + CHUNK))
        def _():
            xb = x_ref[base:base + CHUNK, :]
            h1 = _gelu(lax.dot_general(xb, w1, (((1,), (1,)), ((), ())),
                                       preferred_element_type=jnp.float32)
                       + b1)
            y = lax.dot_general(h1, w2, (((1,), (1,)), ((), ())),
                                preferred_element_type=jnp.float32) + b2
            rows = base + lax.broadcasted_iota(jnp.int32, (CHUNK, 1), 0)
            msk = (rows >= s) & (rows < e)
            o_ref[base:base + CHUNK, :] = (
                o_ref[base:base + CHUNK, :]
                + jnp.where(msk, y - cj, jnp.float32(0.0)))


def _ffn_call(sinfo, sorted_x, w1, b1, w2, b2):
    return pl.pallas_call(
        _ffn_body,
        grid=(NE,),
        in_specs=[
            pl.BlockSpec(memory_space=pltpu.SMEM),
            pl.BlockSpec((NPAD, D), lambda j: (0, 0)),
            pl.BlockSpec((1, HD, D), lambda j: (j, 0, 0)),
            pl.BlockSpec((1, 1, HD), lambda j: (j, 0, 0)),
            pl.BlockSpec((1, D, HD), lambda j: (j, 0, 0)),
            pl.BlockSpec((1, 1, D), lambda j: (j, 0, 0)),
        ],
        out_specs=pl.BlockSpec((NPAD, D), lambda j: (0, 0)),
        out_shape=jax.ShapeDtypeStruct((NPAD, D), jnp.float32),
    )(sinfo, sorted_x, w1, b1.reshape(NE, 1, HD), w2, b2.reshape(NE, 1, D))


@functools.lru_cache(maxsize=None)
def _sc_kernels():
    mesh = plsc.VectorSubcoreMesh(core_axis_name="c", subcore_axis_name="s")
    kern = functools.partial(
        pl.kernel, mesh=mesh,
        out_type=jax.ShapeDtypeStruct((NPAD, D), jnp.float32),
        scratch_types=[
            pltpu.VMEM((BPW,), jnp.int32),
            pltpu.VMEM((BPW, D), jnp.float32),
            pltpu.SemaphoreType.DMA,
        ],
    )

    @kern
    def sc_scatter(x_hbm, idx_hbm, out_hbm, idx_v, rows_v, sem):
        wid = lax.axis_index("s") * 2 + lax.axis_index("c")
        base = wid * BPW
        pltpu.sync_copy(idx_hbm.at[pl.ds(base, BPW)], idx_v)
        pltpu.sync_copy(x_hbm.at[pl.ds(base, BPW)], rows_v)
        pltpu.async_copy(rows_v, out_hbm.at[idx_v], sem).wait()

    @kern
    def sc_gather(y_hbm, idx_hbm, out_hbm, idx_v, rows_v, sem):
        wid = lax.axis_index("s") * 2 + lax.axis_index("c")
        base = wid * BPW
        pltpu.sync_copy(idx_hbm.at[pl.ds(base, BPW)], idx_v)
        pltpu.async_copy(y_hbm.at[idx_v], rows_v, sem).wait()
        pltpu.sync_copy(rows_v, out_hbm.at[pl.ds(base, BPW)])

    return sc_scatter, sc_gather


def _resln_body(h_ref, sy_ref, dest_ref, g_ref, b_ref, o_ref):
    pos = lax.broadcasted_iota(jnp.int32, (NTOK, NPAD), 1)
    pt = (pos == dest_ref[...]).astype(jnp.float32)
    y = lax.dot_general(pt, sy_ref[...], (((1,), (0,)), ((), ())),
                        preferred_element_type=jnp.float32)
    o_ref[...] = _ln(h_ref[...] + y, g_ref[...], b_ref[...])


def _resln_call(h, sy, dest, g, b):
    return pl.pallas_call(
        _resln_body,
        out_shape=jax.ShapeDtypeStruct((NTOK, D), jnp.float32),
    )(h, sy, dest, g.reshape(1, D), b.reshape(1, D))


def _head_body(h_ref, g_ref, b_ref, wf_ref, bf_ref, o_ref):
    x = h_ref[:, 0, :]
    x = _ln(x, g_ref[...], b_ref[...])
    o_ref[...] = lax.dot_general(x, wf_ref[...], (((1,), (1,)), ((), ())),
                                 preferred_element_type=jnp.float32) \
        + bf_ref[...]


def _head_call(h, g, b, wf, bf):
    nc = wf.shape[0]
    return pl.pallas_call(
        _head_body,
        out_shape=jax.ShapeDtypeStruct((B, nc), jnp.float32),
    )(h, g.reshape(1, D), b.reshape(1, D), wf, bf.reshape(1, nc))


def kernel(x, patch_W, patch_b, cls_token, pos_embed, qkv_W, qkv_b, out_W,
           out_b, ln1_g, ln1_b, ln2_g, ln2_b, router_W, router_b, e_W1, e_b1,
           e_W2, e_b2, fn_g, fn_b, fc_W, fc_b):
    nb, C, H, W = x.shape
    hp, wp = H // P, W // P
    patches = x.reshape(nb, C, hp, P, wp, P).transpose(0, 1, 2, 4, 3, 5)
    patches = patches.reshape(nb, C, hp * wp, P * P).transpose(0, 2, 1, 3)
    patches = patches.reshape(nb, hp * wp, C * P * P)

    h = _embed_call(patches, patch_W, patch_b, cls_token, pos_embed)

    L = qkv_W.shape[0]
    nkey = jax.random.key(42)
    for l in range(L):
        h = _attn_call(h, qkv_W[l], qkv_b[l], out_W[l], out_b[l],
                       ln1_g[l], ln1_b[l])
        xf = h.reshape(NTOK, D)
        noise = jax.random.normal(jax.random.fold_in(nkey, l), (NTOK, NE),
                                  dtype=jnp.float32) * 0.01
        dest, sinfo, sorted_x = _route_call(xf, router_W[l], router_b[l],
                                            noise)
        sorted_y = _ffn_call(sinfo.reshape(2 * NE), sorted_x,
                             e_W1[l], e_b1[l], e_W2[l], e_b2[l])
        h = _resln_call(xf, sorted_y, dest, ln2_g[l],
                        ln2_b[l]).reshape(B, S, D)

    return _head_call(h, fn_g, fn_b, fc_W, fc_b)


# speedup vs baseline: 2.2798x; 1.1614x over previous
import functools
import math

import jax
import jax.numpy as jnp
from jax import lax
from jax.experimental import pallas as pl
from jax.experimental.pallas import tpu as pltpu
from jax.experimental.pallas import tpu_sc as plsc

P = 16
NH = 12
NE = 8
D = 384
HD = 1536
S = 197
B = 8
NTOK = B * S
NW = 32
BPW = 56
NPAD = NW * BPW
CHUNK = 256
NCHUNK = NPAD // CHUNK


def _gelu(x):
    return 0.5 * x * (1.0 + lax.erf(x / jnp.sqrt(jnp.float32(2.0))))


def _ln(x, g, b):
    m = jnp.mean(x, axis=-1, keepdims=True)
    v = jnp.mean((x - m) ** 2, axis=-1, keepdims=True)
    return (x - m) / jnp.sqrt(v + 1e-5) * g + b


def _embed_body(p_ref, w_ref, b_ref, cls_ref, pos_ref, o_ref):
    p = p_ref[0]
    t = lax.dot_general(p, w_ref[...], (((1,), (1,)), ((), ())),
                        preferred_element_type=jnp.float32)
    t = t + b_ref[...] + pos_ref[1:, :]
    o_ref[0, 0:1, :] = cls_ref[...] + pos_ref[0:1, :]
    o_ref[0, 1:, :] = t


def _embed_call(patches, patch_W, patch_b, cls_token, pos_embed):
    return pl.pallas_call(
        _embed_body,
        grid=(B,),
        in_specs=[
            pl.BlockSpec((1, 196, 768), lambda i: (i, 0, 0)),
            pl.BlockSpec((D, 768), lambda i: (0, 0)),
            pl.BlockSpec((1, D), lambda i: (0, 0)),
            pl.BlockSpec((1, D), lambda i: (0, 0)),
            pl.BlockSpec((S, D), lambda i: (0, 0)),
        ],
        out_specs=pl.BlockSpec((1, S, D), lambda i: (i, 0, 0)),
        out_shape=jax.ShapeDtypeStruct((B, S, D), jnp.float32),
    )(patches, patch_W, patch_b.reshape(1, D), cls_token.reshape(1, D),
      pos_embed.reshape(S, D))


def _attn_body(h_ref, wqkv_ref, bqkv_ref, wo_ref, bo_ref, g_ref, be_ref,
               o_ref):
    x = h_ref[0]
    qkv = lax.dot_general(x, wqkv_ref[0], (((1,), (1,)), ((), ())),
                          preferred_element_type=jnp.float32) + bqkv_ref[0]
    hd = D // NH
    scale = jnp.sqrt(jnp.float32(hd))
    heads = []
    for i in range(NH):
        q = qkv[:, i * hd:(i + 1) * hd]
        k = qkv[:, D + i * hd:D + (i + 1) * hd]
        v = qkv[:, 2 * D + i * hd:2 * D + (i + 1) * hd]
        s = lax.dot_general(q, k, (((1,), (1,)), ((), ())),
                            preferred_element_type=jnp.float32) / scale
        m = jnp.max(s, axis=1, keepdims=True)
        e = jnp.exp(s - m)
        a = e / jnp.sum(e, axis=1, keepdims=True)
        heads.append(lax.dot_general(a, v, (((1,), (0,)), ((), ())),
                                     preferred_element_type=jnp.float32))
    o = jnp.concatenate(heads, axis=1)
    ao = lax.dot_general(o, wo_ref[0], (((1,), (1,)), ((), ())),
                         preferred_element_type=jnp.float32) + bo_ref[0]
    o_ref[0] = _ln(x + ao, g_ref[0], be_ref[0])


def _attn_call(l, h, qkv_W, qkv_b, out_W, out_b, ln1_g, ln1_b):
    L = qkv_W.shape[0]
    return pl.pallas_call(
        _attn_body,
        grid=(B,),
        in_specs=[
            pl.BlockSpec((1, S, D), lambda i: (i, 0, 0)),
            pl.BlockSpec((1, 3 * D, D), lambda i: (l, 0, 0)),
            pl.BlockSpec((1, 1, 3 * D), lambda i: (l, 0, 0)),
            pl.BlockSpec((1, D, D), lambda i: (l, 0, 0)),
            pl.BlockSpec((1, 1, D), lambda i: (l, 0, 0)),
            pl.BlockSpec((1, 1, D), lambda i: (l, 0, 0)),
            pl.BlockSpec((1, 1, D), lambda i: (l, 0, 0)),
        ],
        out_specs=pl.BlockSpec((1, S, D), lambda i: (i, 0, 0)),
        out_shape=jax.ShapeDtypeStruct((B, S, D), jnp.float32),
    )(h, qkv_W, qkv_b.reshape(L, 1, 3 * D), out_W, out_b.reshape(L, 1, D),
      ln1_g.reshape(L, 1, D), ln1_b.reshape(L, 1, D))


def _route_body(xf_ref, rw_ref, rb_ref, nz_ref, dest_ref, eidx_ref,
                sinfo_ref, sx_ref):
    xf = xf_ref[...]
    logits = lax.dot_general(xf, rw_ref[0], (((1,), (1,)), ((), ())),
                             preferred_element_type=jnp.float32)
    logits = logits + rb_ref[0] + nz_ref[...]
    mx = jnp.max(logits, axis=1, keepdims=True)
    lane = lax.broadcasted_iota(jnp.int32, (NTOK, NE), 1)
    eidx = jnp.min(jnp.where(logits == mx, lane, NE), axis=1, keepdims=True)
    onehot = (lane == eidx).astype(jnp.float32)
    counts = jnp.sum(onehot, axis=0, keepdims=True)
    r8 = lax.broadcasted_iota(jnp.int32, (NE, NE), 0)
    c8 = lax.broadcasted_iota(jnp.int32, (NE, NE), 1)
    tstrict = (r8 < c8).astype(jnp.float32)
    starts = lax.dot_general(counts, tstrict, (((1,), (0,)), ((), ())),
                             preferred_element_type=jnp.float32)
    cum = onehot
    shift = 1
    while shift < NTOK:
        z = jnp.zeros((shift, NE), jnp.float32)
        cum = cum + jnp.concatenate([z, cum[:NTOK - shift, :]], axis=0)
        shift *= 2
    rank = jnp.sum(cum * onehot, axis=1, keepdims=True)
    sbase = jnp.sum(starts * onehot, axis=1, keepdims=True)
    dest = (sbase + rank - 1.0).astype(jnp.int32)
    dest_ref[...] = dest
    eidx_ref[...] = eidx
    ends = starts + counts
    sinfo_ref[...] = jnp.concatenate([starts, ends], axis=1).astype(jnp.int32)
    pos = lax.broadcasted_iota(jnp.int32, (NTOK, NPAD), 1)
    pt = (pos == dest).astype(jnp.float32)
    sx_ref[...] = lax.dot_general(pt, xf, (((0,), (0,)), ((), ())),
                                  preferred_element_type=jnp.float32)


def _route_call(l, xf, router_W, router_b, noise):
    L = router_W.shape[0]
    return pl.pallas_call(
        _route_body,
        grid=(1,),
        in_specs=[
            pl.BlockSpec((NTOK, D), lambda i: (0, 0)),
            pl.BlockSpec((1, NE, D), lambda i: (l, 0, 0)),
            pl.BlockSpec((1, 1, NE), lambda i: (l, 0, 0)),
            pl.BlockSpec((NTOK, NE), lambda i: (0, 0)),
        ],
        out_specs=(
            pl.BlockSpec((NTOK, 1), lambda i: (0, 0)),
            pl.BlockSpec((NTOK, 1), lambda i: (0, 0)),
            pl.BlockSpec((1, 2 * NE), lambda i: (0, 0)),
            pl.BlockSpec((NPAD, D), lambda i: (0, 0)),
        ),
        out_shape=(jax.ShapeDtypeStruct((NTOK, 1), jnp.int32),
                   jax.ShapeDtypeStruct((NTOK, 1), jnp.int32),
                   jax.ShapeDtypeStruct((1, 2 * NE), jnp.int32),
                   jax.ShapeDtypeStruct((NPAD, D), jnp.float32)),
    )(xf, router_W, router_b.reshape(L, 1, NE), noise)


def _ffn_body(sinfo_ref, x_ref, w1_ref, b1_ref, w2_ref, b2_ref, o_ref,
              c_ref):
    j = pl.program_id(0)
    w1 = w1_ref[0, 0]
    w2 = w2_ref[0, 0]
    b1 = b1_ref[0, 0]
    b2 = b2_ref[0, 0]
    c_ref[0] = lax.dot_general(_gelu(b1), w2, (((1,), (1,)), ((), ())),
                               preferred_element_type=jnp.float32) + b2

    @pl.when(j == 0)
    def _():
        o_ref[...] = jnp.zeros((NPAD, D), jnp.float32)

    s = sinfo_ref[j]
    e = sinfo_ref[NE + j]
    for c in range(NCHUNK):
        base = c * CHUNK

        @pl.when((e > base) & (s < base + CHUNK))
        def _():
            xb = x_ref[base:base + CHUNK, :]
            h1 = _gelu(lax.dot_general(xb, w1, (((1,), (1,)), ((), ())),
                                       preferred_element_type=jnp.float32)
                       + b1)
            y = lax.dot_general(h1, w2, (((1,), (1,)), ((), ())),
                                preferred_element_type=jnp.float32) + b2
            rows = base + lax.broadcasted_iota(jnp.int32, (CHUNK, 1), 0)
            msk = (rows >= s) & (rows < e)
            o_ref[base:base + CHUNK, :] = jnp.where(
                msk, y, o_ref[base:base + CHUNK, :])


def _ffn_call(l, sinfo, sorted_x, e_W1, e_b1, e_W2, e_b2):
    L = e_W1.shape[0]
    return pl.pallas_call(
        _ffn_body,
        grid=(NE,),
        in_specs=[
            pl.BlockSpec(memory_space=pltpu.SMEM),
            pl.BlockSpec((NPAD, D), lambda j: (0, 0)),
            pl.BlockSpec((1, 1, HD, D), lambda j: (l, j, 0, 0)),
            pl.BlockSpec((1, 1, 1, HD), lambda j: (l, j, 0, 0)),
            pl.BlockSpec((1, 1, D, HD), lambda j: (l, j, 0, 0)),
            pl.BlockSpec((1, 1, 1, D), lambda j: (l, j, 0, 0)),
        ],
        out_specs=(pl.BlockSpec((NPAD, D), lambda j: (0, 0)),
                   pl.BlockSpec((1, 1, D), lambda j: (j, 0, 0))),
        out_shape=(jax.ShapeDtypeStruct((NPAD, D), jnp.float32),
                   jax.ShapeDtypeStruct((NE, 1, D), jnp.float32)),
    )(sinfo, sorted_x, e_W1, e_b1.reshape(L, NE, 1, HD), e_W2,
      e_b2.reshape(L, NE, 1, D))


def _resln_body(h_ref, sy_ref, dest_ref, eidx_ref, c_ref, g_ref, b_ref,
                o_ref):
    pos = lax.broadcasted_iota(jnp.int32, (NTOK, NPAD), 1)
    pt = (pos == dest_ref[...]).astype(jnp.float32)
    y = lax.dot_general(pt, sy_ref[...], (((1,), (0,)), ((), ())),
                        preferred_element_type=jnp.float32)
    lane = lax.broadcasted_iota(jnp.int32, (NTOK, NE), 1)
    onehot = (lane == eidx_ref[...]).astype(jnp.float32)
    call = c_ref[...]
    csum = jnp.sum(call, axis=0, keepdims=True)
    csel = lax.dot_general(onehot, call, (((1,), (0,)), ((), ())),
                           preferred_element_type=jnp.float32)
    y = y + (csum - csel)
    o_ref[...] = _ln(h_ref[...] + y, g_ref[0], b_ref[0])


def _resln_call(l, h, sy, dest, eidx, call, ln2_g, ln2_b):
    L = ln2_g.shape[0]
    return pl.pallas_call(
        _resln_body,
        grid=(1,),
        in_specs=[
            pl.BlockSpec((NTOK, D), lambda i: (0, 0)),
            pl.BlockSpec((NPAD, D), lambda i: (0, 0)),
            pl.BlockSpec((NTOK, 1), lambda i: (0, 0)),
            pl.BlockSpec((NTOK, 1), lambda i: (0, 0)),
            pl.BlockSpec((NE, D), lambda i: (0, 0)),
            pl.BlockSpec((1, 1, D), lambda i: (l, 0, 0)),
            pl.BlockSpec((1, 1, D), lambda i: (l, 0, 0)),
        ],
        out_specs=pl.BlockSpec((NTOK, D), lambda i: (0, 0)),
        out_shape=jax.ShapeDtypeStruct((NTOK, D), jnp.float32),
    )(h, sy, dest, eidx, call, ln2_g.reshape(L, 1, D), ln2_b.reshape(L, 1, D))


def _head_body(hf_ref, g_ref, b_ref, wf_ref, bf_ref, o_ref):
    x = jnp.concatenate([hf_ref[b * S:b * S + 1, :] for b in range(B)],
                        axis=0)
    x = _ln(x, g_ref[...], b_ref[...])
    o_ref[...] = lax.dot_general(x, wf_ref[...], (((1,), (1,)), ((), ())),
                                 preferred_element_type=jnp.float32) \
        + bf_ref[...]


def _head_call(h, g, b, wf, bf):
    nc = wf.shape[0]
    return pl.pallas_call(
        _head_body,
        out_shape=jax.ShapeDtypeStruct((B, nc), jnp.float32),
    )(h, g.reshape(1, D), b.reshape(1, D), wf, bf.reshape(1, nc))


@functools.lru_cache(maxsize=None)
def _sc_kernels():
    mesh = plsc.VectorSubcoreMesh(core_axis_name="c", subcore_axis_name="s")
    kern = functools.partial(
        pl.kernel, mesh=mesh,
        out_type=jax.ShapeDtypeStruct((NPAD, D), jnp.float32),
        scratch_types=[
            pltpu.VMEM((BPW,), jnp.int32),
            pltpu.VMEM((BPW, D), jnp.float32),
            pltpu.SemaphoreType.DMA,
        ],
    )

    @kern
    def sc_scatter(x_hbm, idx_hbm, out_hbm, idx_v, rows_v, sem):
        wid = lax.axis_index("s") * 2 + lax.axis_index("c")
        base = wid * BPW
        pltpu.sync_copy(idx_hbm.at[pl.ds(base, BPW)], idx_v)
        pltpu.sync_copy(x_hbm.at[pl.ds(base, BPW)], rows_v)
        pltpu.async_copy(rows_v, out_hbm.at[idx_v], sem).wait()

    @kern
    def sc_gather(y_hbm, idx_hbm, out_hbm, idx_v, rows_v, sem):
        wid = lax.axis_index("s") * 2 + lax.axis_index("c")
        base = wid * BPW
        pltpu.sync_copy(idx_hbm.at[pl.ds(base, BPW)], idx_v)
        pltpu.async_copy(y_hbm.at[idx_v], rows_v, sem).wait()
        pltpu.sync_copy(rows_v, out_hbm.at[pl.ds(base, BPW)])

    return sc_scatter, sc_gather


def kernel(x, patch_W, patch_b, cls_token, pos_embed, qkv_W, qkv_b, out_W,
           out_b, ln1_g, ln1_b, ln2_g, ln2_b, router_W, router_b, e_W1, e_b1,
           e_W2, e_b2, fn_g, fn_b, fc_W, fc_b):
    nb, C, H, W = x.shape
    hp, wp = H // P, W // P
    patches = x.reshape(nb, C, hp, P, wp, P).transpose(0, 1, 2, 4, 3, 5)
    patches = patches.reshape(nb, C, hp * wp, P * P).transpose(0, 2, 1, 3)
    patches = patches.reshape(nb, hp * wp, C * P * P)

    h = _embed_call(patches, patch_W, patch_b, cls_token, pos_embed)

    L = qkv_W.shape[0]
    nkey = jax.random.key(42)
    xf = None
    for l in range(L):
        h = _attn_call(l, h, qkv_W, qkv_b, out_W, out_b, ln1_g, ln1_b)
        xf = h.reshape(NTOK, D)
        noise = jax.random.normal(jax.random.fold_in(nkey, l), (NTOK, NE),
                                  dtype=jnp.float32) * 0.01
        dest, eidx, sinfo, sorted_x = _route_call(l, xf, router_W, router_b,
                                                  noise)
        sorted_y, call = _ffn_call(l, sinfo.reshape(2 * NE), sorted_x,
                                   e_W1, e_b1, e_W2, e_b2)
        xf = _resln_call(l, xf, sorted_y, dest, eidx, call.reshape(NE, D),
                         ln2_g, ln2_b)
        h = xf.reshape(B, S, D)

    return _head_call(xf, fn_g, fn_b, fc_W, fc_b)


# speedup vs baseline: 2.9390x; 1.2892x over previous
import functools
import math

import jax
import jax.numpy as jnp
from jax import lax
from jax.experimental import pallas as pl
from jax.experimental.pallas import tpu as pltpu
from jax.experimental.pallas import tpu_sc as plsc

P = 16
NH = 12
NE = 8
D = 384
HD = 1536
S = 197
B = 8
NTOK = B * S
NW = 32
BPW = 56
NPAD = NW * BPW
CHUNK = 256
NCHUNK = NPAD // CHUNK


def _gelu(x):
    return 0.5 * x * (1.0 + lax.erf(x / jnp.sqrt(jnp.float32(2.0))))


def _ln(x, g, b):
    m = jnp.mean(x, axis=-1, keepdims=True)
    v = jnp.mean((x - m) ** 2, axis=-1, keepdims=True)
    return (x - m) / jnp.sqrt(v + 1e-5) * g + b


def _embed_body(x_ref, w_ref, b_ref, cls_ref, pos_ref, o_ref):
    pieces = []
    nch = x_ref.shape[1]
    for ch in range(nch):
        for i in range(P):
            pieces.append(x_ref[0, ch, :, i, :, :].reshape(196, P))
    p = jnp.concatenate(pieces, axis=1)
    t = lax.dot_general(p, w_ref[...], (((1,), (1,)), ((), ())),
                        preferred_element_type=jnp.float32)
    t = t + b_ref[...] + pos_ref[1:, :]
    o_ref[0, 0:1, :] = cls_ref[...] + pos_ref[0:1, :]
    o_ref[0, 1:, :] = t


def _embed_call(x6, patch_W, patch_b, cls_token, pos_embed):
    C = x6.shape[1]
    return pl.pallas_call(
        _embed_body,
        grid=(B,),
        in_specs=[
            pl.BlockSpec((1, C, 14, P, 14, P), lambda i: (i, 0, 0, 0, 0, 0)),
            pl.BlockSpec((D, 768), lambda i: (0, 0)),
            pl.BlockSpec((1, D), lambda i: (0, 0)),
            pl.BlockSpec((1, D), lambda i: (0, 0)),
            pl.BlockSpec((S, D), lambda i: (0, 0)),
        ],
        out_specs=pl.BlockSpec((1, S, D), lambda i: (i, 0, 0)),
        out_shape=jax.ShapeDtypeStruct((B, S, D), jnp.float32),
    )(x6, patch_W, patch_b.reshape(1, D), cls_token.reshape(1, D),
      pos_embed.reshape(S, D))


def _attn_body(h_ref, wqkv_ref, bqkv_ref, wo_ref, bo_ref, g_ref, be_ref,
               o_ref):
    x = h_ref[0]
    qkv = lax.dot_general(x, wqkv_ref[0], (((1,), (1,)), ((), ())),
                          preferred_element_type=jnp.float32) + bqkv_ref[0]
    hd = D // NH
    scale = jnp.sqrt(jnp.float32(hd))
    heads = []
    for i in range(NH):
        q = qkv[:, i * hd:(i + 1) * hd]
        k = qkv[:, D + i * hd:D + (i + 1) * hd]
        v = qkv[:, 2 * D + i * hd:2 * D + (i + 1) * hd]
        s = lax.dot_general(q, k, (((1,), (1,)), ((), ())),
                            preferred_element_type=jnp.float32) / scale
        m = jnp.max(s, axis=1, keepdims=True)
        e = jnp.exp(s - m)
        a = e / jnp.sum(e, axis=1, keepdims=True)
        heads.append(lax.dot_general(a, v, (((1,), (0,)), ((), ())),
                                     preferred_element_type=jnp.float32))
    o = jnp.concatenate(heads, axis=1)
    ao = lax.dot_general(o, wo_ref[0], (((1,), (1,)), ((), ())),
                         preferred_element_type=jnp.float32) + bo_ref[0]
    o_ref[0] = _ln(x + ao, g_ref[0], be_ref[0])


def _attn_call(l, h, qkv_W, qkv_b, out_W, out_b, ln1_g, ln1_b):
    L = qkv_W.shape[0]
    return pl.pallas_call(
        _attn_body,
        grid=(B,),
        in_specs=[
            pl.BlockSpec((1, S, D), lambda i: (i, 0, 0)),
            pl.BlockSpec((1, 3 * D, D), lambda i: (l, 0, 0)),
            pl.BlockSpec((1, 1, 3 * D), lambda i: (l, 0, 0)),
            pl.BlockSpec((1, D, D), lambda i: (l, 0, 0)),
            pl.BlockSpec((1, 1, D), lambda i: (l, 0, 0)),
            pl.BlockSpec((1, 1, D), lambda i: (l, 0, 0)),
            pl.BlockSpec((1, 1, D), lambda i: (l, 0, 0)),
        ],
        out_specs=pl.BlockSpec((1, S, D), lambda i: (i, 0, 0)),
        out_shape=jax.ShapeDtypeStruct((B, S, D), jnp.float32),
    )(h, qkv_W, qkv_b.reshape(L, 1, 3 * D), out_W, out_b.reshape(L, 1, D),
      ln1_g.reshape(L, 1, D), ln1_b.reshape(L, 1, D))


def _route_body(xf_ref, rw_ref, rb_ref, nz_ref, dest_ref, eidx_ref,
                sinfo_ref, sx_ref):
    xf = xf_ref[...]
    logits = lax.dot_general(xf, rw_ref[0], (((1,), (1,)), ((), ())),
                             preferred_element_type=jnp.float32)
    logits = logits + rb_ref[0] + nz_ref[...]
    mx = jnp.max(logits, axis=1, keepdims=True)
    lane = lax.broadcasted_iota(jnp.int32, (NTOK, NE), 1)
    eidx = jnp.min(jnp.where(logits == mx, lane, NE), axis=1, keepdims=True)
    onehot = (lane == eidx).astype(jnp.float32)
    counts = jnp.sum(onehot, axis=0, keepdims=True)
    r8 = lax.broadcasted_iota(jnp.int32, (NE, NE), 0)
    c8 = lax.broadcasted_iota(jnp.int32, (NE, NE), 1)
    tstrict = (r8 < c8).astype(jnp.float32)
    starts = lax.dot_general(counts, tstrict, (((1,), (0,)), ((), ())),
                             preferred_element_type=jnp.float32)
    cum = onehot
    shift = 1
    while shift < NTOK:
        z = jnp.zeros((shift, NE), jnp.float32)
        cum = cum + jnp.concatenate([z, cum[:NTOK - shift, :]], axis=0)
        shift *= 2
    rank = jnp.sum(cum * onehot, axis=1, keepdims=True)
    sbase = jnp.sum(starts * onehot, axis=1, keepdims=True)
    dest = (sbase + rank - 1.0).astype(jnp.int32)
    dest_ref[...] = dest
    eidx_ref[...] = eidx
    ends = starts + counts
    sinfo_ref[...] = jnp.concatenate([starts, ends], axis=1).astype(jnp.int32)
    pos = lax.broadcasted_iota(jnp.int32, (NTOK, NPAD), 1)
    pt = (pos == dest).astype(jnp.float32)
    sx_ref[...] = lax.dot_general(pt, xf, (((0,), (0,)), ((), ())),
                                  preferred_element_type=jnp.float32)


def _route_call(l, xf, router_W, router_b, noise):
    L = router_W.shape[0]
    return pl.pallas_call(
        _route_body,
        grid=(1,),
        in_specs=[
            pl.BlockSpec((NTOK, D), lambda i: (0, 0)),
            pl.BlockSpec((1, NE, D), lambda i: (l, 0, 0)),
            pl.BlockSpec((1, 1, NE), lambda i: (l, 0, 0)),
            pl.BlockSpec((NTOK, NE), lambda i: (0, 0)),
        ],
        out_specs=(
            pl.BlockSpec((NTOK, 1), lambda i: (0, 0)),
            pl.BlockSpec((NTOK, 1), lambda i: (0, 0)),
            pl.BlockSpec((1, 2 * NE), lambda i: (0, 0)),
            pl.BlockSpec((NPAD, D), lambda i: (0, 0)),
        ),
        out_shape=(jax.ShapeDtypeStruct((NTOK, 1), jnp.int32),
                   jax.ShapeDtypeStruct((NTOK, 1), jnp.int32),
                   jax.ShapeDtypeStruct((1, 2 * NE), jnp.int32),
                   jax.ShapeDtypeStruct((NPAD, D), jnp.float32)),
    )(xf, router_W, router_b.reshape(L, 1, NE), noise)


def _ffn_body(sinfo_ref, x_ref, w1_ref, b1_ref, w2_ref, b2_ref, o_ref,
              c_ref):
    j = pl.program_id(0)
    w1 = w1_ref[0, 0]
    w2 = w2_ref[0, 0]
    b1 = b1_ref[0, 0]
    b2 = b2_ref[0, 0]
    c_ref[0] = lax.dot_general(_gelu(b1), w2, (((1,), (1,)), ((), ())),
                               preferred_element_type=jnp.float32) + b2

    @pl.when(j == 0)
    def _():
        o_ref[...] = jnp.zeros((NPAD, D), jnp.float32)

    s = sinfo_ref[j]
    e = sinfo_ref[NE + j]
    for c in range(NCHUNK):
        base = c * CHUNK

        @pl.when((e > base) & (s < base + CHUNK))
        def _():
            xb = x_ref[base:base + CHUNK, :]
            h1 = _gelu(lax.dot_general(xb, w1, (((1,), (1,)), ((), ())),
                                       preferred_element_type=jnp.float32)
                       + b1)
            y = lax.dot_general(h1, w2, (((1,), (1,)), ((), ())),
                                preferred_element_type=jnp.float32) + b2
            rows = base + lax.broadcasted_iota(jnp.int32, (CHUNK, 1), 0)
            msk = (rows >= s) & (rows < e)
            o_ref[base:base + CHUNK, :] = jnp.where(
                msk, y, o_ref[base:base + CHUNK, :])


def _ffn_call(l, sinfo, sorted_x, e_W1, e_b1, e_W2, e_b2):
    L = e_W1.shape[0]
    return pl.pallas_call(
        _ffn_body,
        grid=(NE,),
        in_specs=[
            pl.BlockSpec(memory_space=pltpu.SMEM),
            pl.BlockSpec((NPAD, D), lambda j: (0, 0)),
            pl.BlockSpec((1, 1, HD, D), lambda j: (l, j, 0, 0)),
            pl.BlockSpec((1, 1, 1, HD), lambda j: (l, j, 0, 0)),
            pl.BlockSpec((1, 1, D, HD), lambda j: (l, j, 0, 0)),
            pl.BlockSpec((1, 1, 1, D), lambda j: (l, j, 0, 0)),
        ],
        out_specs=(pl.BlockSpec((NPAD, D), lambda j: (0, 0)),
                   pl.BlockSpec((1, 1, D), lambda j: (j, 0, 0))),
        out_shape=(jax.ShapeDtypeStruct((NPAD, D), jnp.float32),
                   jax.ShapeDtypeStruct((NE, 1, D), jnp.float32)),
    )(sinfo, sorted_x, e_W1, e_b1.reshape(L, NE, 1, HD), e_W2,
      e_b2.reshape(L, NE, 1, D))


def _resln_body(h_ref, sy_ref, dest_ref, eidx_ref, c_ref, g_ref, b_ref,
                o_ref):
    pos = lax.broadcasted_iota(jnp.int32, (NTOK, NPAD), 1)
    pt = (pos == dest_ref[...]).astype(jnp.float32)
    y = lax.dot_general(pt, sy_ref[...], (((1,), (0,)), ((), ())),
                        preferred_element_type=jnp.float32)
    lane = lax.broadcasted_iota(jnp.int32, (NTOK, NE), 1)
    onehot = (lane == eidx_ref[...]).astype(jnp.float32)
    call = c_ref[...]
    csum = jnp.sum(call, axis=0, keepdims=True)
    csel = lax.dot_general(onehot, call, (((1,), (0,)), ((), ())),
                           preferred_element_type=jnp.float32)
    y = y + (csum - csel)
    o_ref[...] = _ln(h_ref[...] + y, g_ref[0], b_ref[0])


def _resln_call(l, h, sy, dest, eidx, call, ln2_g, ln2_b):
    L = ln2_g.shape[0]
    return pl.pallas_call(
        _resln_body,
        grid=(1,),
        in_specs=[
            pl.BlockSpec((NTOK, D), lambda i: (0, 0)),
            pl.BlockSpec((NPAD, D), lambda i: (0, 0)),
            pl.BlockSpec((NTOK, 1), lambda i: (0, 0)),
            pl.BlockSpec((NTOK, 1), lambda i: (0, 0)),
            pl.BlockSpec((NE, D), lambda i: (0, 0)),
            pl.BlockSpec((1, 1, D), lambda i: (l, 0, 0)),
            pl.BlockSpec((1, 1, D), lambda i: (l, 0, 0)),
        ],
        out_specs=pl.BlockSpec((NTOK, D), lambda i: (0, 0)),
        out_shape=jax.ShapeDtypeStruct((NTOK, D), jnp.float32),
    )(h, sy, dest, eidx, call, ln2_g.reshape(L, 1, D), ln2_b.reshape(L, 1, D))


def _head_body(hf_ref, g_ref, b_ref, wf_ref, bf_ref, o_ref):
    x = jnp.concatenate([hf_ref[b * S:b * S + 1, :] for b in range(B)],
                        axis=0)
    x = _ln(x, g_ref[...], b_ref[...])
    o_ref[...] = lax.dot_general(x, wf_ref[...], (((1,), (1,)), ((), ())),
                                 preferred_element_type=jnp.float32) \
        + bf_ref[...]


def _head_call(h, g, b, wf, bf):
    nc = wf.shape[0]
    return pl.pallas_call(
        _head_body,
        out_shape=jax.ShapeDtypeStruct((B, nc), jnp.float32),
    )(h, g.reshape(1, D), b.reshape(1, D), wf, bf.reshape(1, nc))


@functools.lru_cache(maxsize=None)
def _sc_kernels():
    mesh = plsc.VectorSubcoreMesh(core_axis_name="c", subcore_axis_name="s")
    kern = functools.partial(
        pl.kernel, mesh=mesh,
        out_type=jax.ShapeDtypeStruct((NPAD, D), jnp.float32),
        scratch_types=[
            pltpu.VMEM((BPW,), jnp.int32),
            pltpu.VMEM((BPW, D), jnp.float32),
            pltpu.SemaphoreType.DMA,
        ],
    )

    @kern
    def sc_scatter(x_hbm, idx_hbm, out_hbm, idx_v, rows_v, sem):
        wid = lax.axis_index("s") * 2 + lax.axis_index("c")
        base = wid * BPW
        pltpu.sync_copy(idx_hbm.at[pl.ds(base, BPW)], idx_v)
        pltpu.sync_copy(x_hbm.at[pl.ds(base, BPW)], rows_v)
        pltpu.async_copy(rows_v, out_hbm.at[idx_v], sem).wait()

    @kern
    def sc_gather(y_hbm, idx_hbm, out_hbm, idx_v, rows_v, sem):
        wid = lax.axis_index("s") * 2 + lax.axis_index("c")
        base = wid * BPW
        pltpu.sync_copy(idx_hbm.at[pl.ds(base, BPW)], idx_v)
        pltpu.async_copy(y_hbm.at[idx_v], rows_v, sem).wait()
        pltpu.sync_copy(rows_v, out_hbm.at[pl.ds(base, BPW)])

    return sc_scatter, sc_gather


def kernel(x, patch_W, patch_b, cls_token, pos_embed, qkv_W, qkv_b, out_W,
           out_b, ln1_g, ln1_b, ln2_g, ln2_b, router_W, router_b, e_W1, e_b1,
           e_W2, e_b2, fn_g, fn_b, fc_W, fc_b):
    nb, C, H, W = x.shape
    hp, wp = H // P, W // P
    x6 = x.reshape(nb, C, hp, P, wp, P)

    h = _embed_call(x6, patch_W, patch_b, cls_token, pos_embed)

    L = qkv_W.shape[0]
    nkey = jax.random.key(42)
    xf = None
    for l in range(L):
        h = _attn_call(l, h, qkv_W, qkv_b, out_W, out_b, ln1_g, ln1_b)
        xf = h.reshape(NTOK, D)
        noise = jax.random.normal(jax.random.fold_in(nkey, l), (NTOK, NE),
                                  dtype=jnp.float32) * 0.01
        dest, eidx, sinfo, sorted_x = _route_call(l, xf, router_W, router_b,
                                                  noise)
        sorted_y, call = _ffn_call(l, sinfo.reshape(2 * NE), sorted_x,
                                   e_W1, e_b1, e_W2, e_b2)
        xf = _resln_call(l, xf, sorted_y, dest, eidx, call.reshape(NE, D),
                         ln2_g, ln2_b)
        h = xf.reshape(B, S, D)

    return _head_call(xf, fn_g, fn_b, fc_W, fc_b)
